# baseline (speedup 1.0000x reference)
import functools

import jax
import jax.numpy as jnp
from jax import lax
from jax.experimental import pallas as pl
from jax.experimental.pallas import tpu as pltpu
from jax.experimental.pallas import tpu_sc as plsc

BS = 1024
IN = 128
OUT = 100000
K = 4096
T = 0.07
MOMENTUM = 0.5

HB = 512
TN = 2048
NT = 49
NCHT = TN // 128
NCH = NT * NCHT
KP = 33 * 128
NEG = -1e30


def _sim_body(x_ref, m_ref, out_ref, cmax_ref):
    t = pl.program_id(0)
    tile = jax.lax.dot_general(
        x_ref[...], m_ref[...], (((1,), (1,)), ((), ())),
        preferred_element_type=jnp.float32,
        precision=jax.lax.Precision.DEFAULT)
    col = jax.lax.broadcasted_iota(jnp.int32, (HB, TN), 1) + t * TN
    tile = jnp.where(col < OUT, tile, NEG)
    out_ref[...] = tile.reshape(HB * NCHT, 128)
    for c in range(NCHT):
        cmax_ref[0, c, :] = jnp.max(tile[:, c * 128:(c + 1) * 128], axis=1)


def _similarity(xh, memory):
    return pl.pallas_call(
        _sim_body,
        grid=(NT,),
        in_specs=[
            pl.BlockSpec((HB, IN), lambda t: (0, 0)),
            pl.BlockSpec((TN, IN), lambda t: (t, 0)),
        ],
        out_specs=[
            pl.BlockSpec((HB * NCHT, 128), lambda t: (t, 0)),
            pl.BlockSpec((1, NCHT, HB), lambda t: (t, 0, 0)),
        ],
        out_shape=[
            jax.ShapeDtypeStruct((NT * HB * NCHT, 128), jnp.float32),
            jax.ShapeDtypeStruct((NT, NCHT, HB), jnp.float32),
        ],
    )(xh, memory)


def _topchunk_body(cm_ref, cid_ref):
    v = cm_ref[...].reshape(NCH, HB)
    ii = jax.lax.broadcasted_iota(jnp.int32, (NCH, HB), 0)
    for k in range(32):
        m = jnp.max(v, axis=0)
        sel = jnp.min(jnp.where(v == m[None, :], ii, NCH), axis=0)
        cid_ref[k, :] = sel
        v = jnp.where(ii == sel[None, :], -jnp.inf, v)


def _topchunks(cmax):
    return pl.pallas_call(
        _topchunk_body,
        out_shape=jax.ShapeDtypeStruct((32, HB), jnp.int32),
    )(cmax)


def _topk_body(cand_ref, cols_ref, yd_ref, yi_ref):
    v = cand_ref[...]
    cols = cols_ref[...]
    for k in range(32):
        m = jnp.max(v, axis=1)
        sel = jnp.min(jnp.where(v == m[:, None], cols, jnp.int32(2**30)), axis=1)
        yd_ref[:, k] = m
        yi_ref[:, k] = sel
        v = jnp.where(cols == sel[:, None], -jnp.inf, v)


def _topk(cand, cols):
    return pl.pallas_call(
        _topk_body,
        out_shape=[
            jax.ShapeDtypeStruct((HB, 32), jnp.float32),
            jax.ShapeDtypeStruct((HB, 32), jnp.int32),
        ],
    )(cand, cols)


def _exp_body(nce_ref, e_ref, rs_ref):
    col = jax.lax.broadcasted_iota(jnp.int32, (HB, KP), 1)
    v = jnp.where(col <= K, nce_ref[...], -jnp.inf)
    e = jnp.exp(v * jnp.float32(1.0 / T))
    e_ref[...] = e
    rs_ref[...] = jnp.sum(e, axis=1, keepdims=True)


def _exp_norm(nce_pad):
    return pl.pallas_call(
        _exp_body,
        out_shape=[
            jax.ShapeDtypeStruct((HB, KP), jnp.float32),
            jax.ShapeDtypeStruct((HB, 1), jnp.float32),
        ],
    )(nce_pad)


def _norm_body(my_ref, xw_ref, o_ref):
    w = my_ref[...] * jnp.float32(MOMENTUM) + xw_ref[...] * jnp.float32(1.0 - MOMENTUM)
    n = jnp.maximum(jnp.sqrt(jnp.sum(w * w, axis=1, keepdims=True)), 1e-12)
    o_ref[...] = w / n


def _mix_norm(mem_y, xw):
    return pl.pallas_call(
        _norm_body,
        out_shape=jax.ShapeDtypeStruct((BS, IN), jnp.float32),
    )(mem_y, xw)


NW = 32
ROWS_PER_W = BS // NW
RPW_H = HB // NW
FLAT_H = NT * HB * TN
NADDR = KP // 128


def _sc_mesh():
    return plsc.VectorSubcoreMesh(core_axis_name="c", subcore_axis_name="s")


def _wid():
    return lax.axis_index("s") * 2 + lax.axis_index("c")


def _cand_gather(table, rows3d):
    @functools.partial(
        pl.kernel,
        out_type=jax.ShapeDtypeStruct((HB * 32, 128), jnp.float32),
        mesh=_sc_mesh(),
        scratch_types=[
            pltpu.VMEM((4, 128), jnp.int32),
            pltpu.VMEM((128, 128), jnp.float32),
            pltpu.SemaphoreType.DMA,
        ],
    )
    def k(tab, ridx, out, idx_v, buf_v, sem):
        w = _wid()
        pltpu.sync_copy(ridx.at[w], idx_v)

        def body(s, carry):
            pltpu.async_copy(tab.at[idx_v.at[s]], buf_v, sem).wait()
            pltpu.sync_copy(buf_v, out.at[pl.ds(w * 512 + s * 128, 128)])
            return carry

        lax.fori_loop(0, 4, body, 0)

    return k(table, rows3d)


def _nce_gather(table_flat, idx_flat):
    @functools.partial(
        pl.kernel,
        out_type=jax.ShapeDtypeStruct((HB, KP), jnp.float32),
        mesh=_sc_mesh(),
        scratch_types=[
            pltpu.VMEM((KP,), jnp.int32),
            pltpu.VMEM((NADDR, 128), jnp.int32),
            pltpu.VMEM((KP,), jnp.float32),
            pltpu.SemaphoreType.DMA,
        ],
    )
    def k(tab, idx_hbm, out, col_v, addr_v, val_v, sem):
        w = _wid()

        def row_body(r, carry):
            b = w * RPW_H + r
            pltpu.sync_copy(idx_hbm.at[pl.ds(b * KP, KP)], col_v)

            def addr_chunk(j, c2):
                for o in range(8):
                    col = col_v[pl.ds(j * 128 + o * 16, 16)]
                    t = lax.shift_right_arithmetic(col, 11)
                    cc = lax.bitwise_and(col, TN - 1)
                    f = lax.shift_left(t, 20) + (b * TN + cc)
                    addr_v[j, pl.ds(o * 16, 16)] = f
                return c2

            lax.fori_loop(0, NADDR, addr_chunk, 0)

            def fire(j, c2):
                pltpu.async_copy(
                    tab.at[addr_v.at[j]], val_v.at[pl.ds(j * 128, 128)], sem)
                return c2

            lax.fori_loop(0, NADDR, fire, 0)

            def drain(j, c2):
                pltpu.make_async_copy(
                    tab.at[addr_v.at[j]], val_v.at[pl.ds(j * 128, 128)], sem
                ).wait()
                return c2

            lax.fori_loop(0, NADDR, drain, 0)
            pltpu.sync_copy(val_v, out.at[b])
            return carry

        lax.fori_loop(0, RPW_H, row_body, 0)

    return k(table_flat, idx_flat)


def _label_gather(trainLabel, yi3d):
    @functools.partial(
        pl.kernel,
        out_type=jax.ShapeDtypeStruct((BS * 32,), jnp.int32),
        mesh=_sc_mesh(),
        scratch_types=[
            pltpu.VMEM((8, 128), jnp.int32),
            pltpu.VMEM((128,), jnp.int32),
            pltpu.SemaphoreType.DMA,
        ],
    )
    def k(tab, ridx, out, idx_v, buf_v, sem):
        w = _wid()
        pltpu.sync_copy(ridx.at[w], idx_v)

        def body(s, carry):
            pltpu.async_copy(tab.at[idx_v.at[s]], buf_v, sem).wait()
            pltpu.sync_copy(buf_v, out.at[pl.ds(w * 1024 + s * 128, 128)])
            return carry

        lax.fori_loop(0, 8, body, 0)

    return k(trainLabel, yi3d)


def _update_gathers(memory, x, ysort, wsort):
    @functools.partial(
        pl.kernel,
        out_type=[
            jax.ShapeDtypeStruct((BS, IN), jnp.float32),
            jax.ShapeDtypeStruct((BS, IN), jnp.float32),
        ],
        mesh=_sc_mesh(),
        scratch_types=[
            pltpu.VMEM((ROWS_PER_W,), jnp.int32),
            pltpu.VMEM((ROWS_PER_W, IN), jnp.float32),
            pltpu.SemaphoreType.DMA,
        ],
    )
    def k(mem, xx, ys, ws, out_my, out_xw, idx_v, buf_v, sem):
        w = _wid()
        base = w * ROWS_PER_W
        pltpu.sync_copy(ys.at[pl.ds(base, ROWS_PER_W)], idx_v)
        pltpu.async_copy(mem.at[idx_v], buf_v, sem).wait()
        pltpu.sync_copy(buf_v, out_my.at[pl.ds(base, ROWS_PER_W)])
        pltpu.sync_copy(ws.at[pl.ds(base, ROWS_PER_W)], idx_v)
        pltpu.async_copy(xx.at[idx_v], buf_v, sem).wait()
        pltpu.sync_copy(buf_v, out_xw.at[pl.ds(base, ROWS_PER_W)])

    return k(memory, x, ysort, wsort)


def _update_scatter(new_mem_ref, normed, y):
    @functools.partial(
        pl.kernel,
        out_type=(),
        mesh=_sc_mesh(),
        scratch_types=[
            pltpu.VMEM((ROWS_PER_W,), jnp.int32),
            pltpu.VMEM((ROWS_PER_W, IN), jnp.float32),
            pltpu.SemaphoreType.DMA,
        ],
    )
    def k(nrm, yy, out, idx_v, buf_v, sem):
        w = _wid()
        base = w * ROWS_PER_W
        pltpu.sync_copy(yy.at[pl.ds(base, ROWS_PER_W)], idx_v)
        pltpu.sync_copy(nrm.at[pl.ds(base, ROWS_PER_W)], buf_v)
        pltpu.async_copy(buf_v, out.at[idx_v], sem).wait()

    k(normed, y, new_mem_ref)


def _half_pipeline(xh, idx_flat, memory, harange):
    out_h, cmax = _similarity(xh, memory)

    nce_pad = _nce_gather(out_h.reshape(FLAT_H), idx_flat)

    chunk_ids = _topchunks(cmax)
    cid_t = chunk_ids.T

    tt = cid_t // NCHT
    ci = cid_t % NCHT
    rows = (tt * HB + harange[:, None]) * NCHT + ci
    cand = _cand_gather(out_h, rows.reshape(NW, 4, 128))
    cand = cand.reshape(HB, 32 * 128)
    cols = (cid_t[:, :, None] * 128
            + jnp.arange(128, dtype=jnp.int32)[None, None, :]).reshape(HB, 32 * 128)
    return nce_pad, cand, cols


def kernel(x, target, y, idx, trainLabel, memory):
    iarange = jnp.arange(BS, dtype=jnp.int32)
    winner = jnp.argmax(jnp.where(y[None, :] == y[:, None], iarange[None, :], -1),
                        axis=1).astype(jnp.int32)

    mem_y, xw = _update_gathers(memory, x, y, winner)
    normed = _mix_norm(mem_y, xw)
    mem_for_clone, _ = jax.lax.optimization_barrier((memory, normed))
    new_mem_ref = jax.new_ref(mem_for_clone)
    _update_scatter(new_mem_ref, normed, y)
    new_memory = new_mem_ref[...]

    idx_pad = jnp.pad(idx, ((0, 0), (0, KP - (K + 1)))).reshape(BS * KP)
    idx_flatA = idx_pad[:HB * KP]
    idx_flatB = idx_pad[HB * KP:]

    harange = iarange[:HB]
    nceA, candA, colsA = _half_pipeline(x[:HB], idx_flatA, memory, harange)
    nceB, candB, colsB = _half_pipeline(x[HB:], idx_flatB, memory, harange)

    ydA, yiA = _topk(candA, colsA)
    ydB, yiB = _topk(candB, colsB)
    yd = jnp.concatenate([ydA, ydB], axis=0)
    yi = jnp.concatenate([yiA, yiB], axis=0)
    retrieval = _label_gather(trainLabel, yi.reshape(NW, 8, 128)).reshape(BS, 32)

    eA, rsA = _exp_norm(nceA)
    eB, rsB = _exp_norm(nceB)
    total = jnp.sum(rsA) + jnp.sum(rsB)
    Z = total / jnp.float32(BS * (K + 1)) * jnp.float32(OUT)
    out = jnp.concatenate([eA[:, :K + 1], eB[:, :K + 1]], axis=0) / Z
    probs = (jnp.sum(eA[:, 0] / rsA[:, 0]) + jnp.sum(eB[:, 0] / rsB[:, 0])) / jnp.float32(BS)

    return out, probs, yd, retrieval, new_memory


# speedup vs baseline: 1.0145x; 1.0145x over previous
import functools

import jax
import jax.numpy as jnp
from jax import lax
from jax.experimental import pallas as pl
from jax.experimental.pallas import tpu as pltpu
from jax.experimental.pallas import tpu_sc as plsc

BS = 1024
IN = 128
OUT = 100000
K = 4096
T = 0.07
MOMENTUM = 0.5

HB = 512
TN = 2048
NT = 49
NCHT = TN // 128
NCH = NT * NCHT
KP = 33 * 128
NEG = -1e30


def _sim_body(x_ref, m_ref, out_ref, cmax_ref):
    t = pl.program_id(0)
    tile = jax.lax.dot_general(
        x_ref[...], m_ref[...], (((1,), (1,)), ((), ())),
        preferred_element_type=jnp.float32,
        precision=jax.lax.Precision.DEFAULT)
    col = jax.lax.broadcasted_iota(jnp.int32, (HB, TN), 1) + t * TN
    tile = jnp.where(col < OUT, tile, NEG)
    out_ref[...] = tile.reshape(HB * NCHT, 128)
    for c in range(NCHT):
        cmax_ref[0, c, :] = jnp.max(tile[:, c * 128:(c + 1) * 128], axis=1)


def _similarity(xh, memory):
    return pl.pallas_call(
        _sim_body,
        grid=(NT,),
        in_specs=[
            pl.BlockSpec((HB, IN), lambda t: (0, 0)),
            pl.BlockSpec((TN, IN), lambda t: (t, 0)),
        ],
        out_specs=[
            pl.BlockSpec((HB * NCHT, 128), lambda t: (t, 0)),
            pl.BlockSpec((1, NCHT, HB), lambda t: (t, 0, 0)),
        ],
        out_shape=[
            jax.ShapeDtypeStruct((NT * HB * NCHT, 128), jnp.float32),
            jax.ShapeDtypeStruct((NT, NCHT, HB), jnp.float32),
        ],
    )(xh, memory)


def _topchunk_body(cm_ref, cid_ref, rows_ref):
    v = cm_ref[...].reshape(NCH, HB)
    ii = jax.lax.broadcasted_iota(jnp.int32, (NCH, HB), 0)
    bvec = jax.lax.broadcasted_iota(jnp.int32, (NCH, HB), 1)[0]
    for k in range(32):
        m = jnp.max(v, axis=0)
        sel = jnp.min(jnp.where(v == m[None, :], ii, NCH), axis=0)
        cid_ref[k, :] = sel
        rows_ref[k, :] = ((jax.lax.shift_right_logical(sel, 4) * HB + bvec) * NCHT
                          + jax.lax.bitwise_and(sel, NCHT - 1))
        v = jnp.where(ii == sel[None, :], -jnp.inf, v)


def _topchunks(cmax):
    return pl.pallas_call(
        _topchunk_body,
        out_shape=[
            jax.ShapeDtypeStruct((32, HB), jnp.int32),
            jax.ShapeDtypeStruct((32, HB), jnp.int32),
        ],
    )(cmax)


def _topk_body(cand_ref, cols_ref, yd_ref, yi_ref):
    v = cand_ref[...]
    cols = cols_ref[...]
    for k in range(32):
        m = jnp.max(v, axis=1)
        sel = jnp.min(jnp.where(v == m[:, None], cols, jnp.int32(2**30)), axis=1)
        yd_ref[:, k] = m
        yi_ref[:, k] = sel
        v = jnp.where(cols == sel[:, None], -jnp.inf, v)


def _topk(cand, cols):
    return pl.pallas_call(
        _topk_body,
        out_shape=[
            jax.ShapeDtypeStruct((HB, 32), jnp.float32),
            jax.ShapeDtypeStruct((HB, 32), jnp.int32),
        ],
    )(cand, cols)


def _exp_body(nce_ref, e_ref, rs_ref):
    col = jax.lax.broadcasted_iota(jnp.int32, (HB, KP), 1)
    v = jnp.where(col <= K, nce_ref[...], -jnp.inf)
    e = jnp.exp(v * jnp.float32(1.0 / T))
    e_ref[...] = e
    rs_ref[...] = jnp.sum(e, axis=1, keepdims=True)


def _exp_norm(nce_pad):
    return pl.pallas_call(
        _exp_body,
        out_shape=[
            jax.ShapeDtypeStruct((HB, KP), jnp.float32),
            jax.ShapeDtypeStruct((HB, 1), jnp.float32),
        ],
    )(nce_pad)


def _norm_body(my_ref, xw_ref, o_ref):
    w = my_ref[...] * jnp.float32(MOMENTUM) + xw_ref[...] * jnp.float32(1.0 - MOMENTUM)
    n = jnp.maximum(jnp.sqrt(jnp.sum(w * w, axis=1, keepdims=True)), 1e-12)
    o_ref[...] = w / n


def _mix_norm(mem_y, xw):
    return pl.pallas_call(
        _norm_body,
        out_shape=jax.ShapeDtypeStruct((BS, IN), jnp.float32),
    )(mem_y, xw)


NW = 32
ROWS_PER_W = BS // NW
RPW_H = HB // NW
FLAT_H = NT * HB * TN
NADDR = KP // 128


def _sc_mesh():
    return plsc.VectorSubcoreMesh(core_axis_name="c", subcore_axis_name="s")


def _wid():
    return lax.axis_index("s") * 2 + lax.axis_index("c")


def _cand_gather(table, rowsk3d, pos3d):
    @functools.partial(
        pl.kernel,
        out_type=jax.ShapeDtypeStruct((HB * 32, 128), jnp.float32),
        mesh=_sc_mesh(),
        scratch_types=[
            pltpu.VMEM((4, 128), jnp.int32),
            pltpu.VMEM((4, 128), jnp.int32),
            pltpu.VMEM((128, 128), jnp.float32),
            pltpu.SemaphoreType.DMA,
        ],
    )
    def k(tab, ridx, pidx, out, idx_v, pos_v, buf_v, sem):
        w = _wid()
        pltpu.sync_copy(ridx.at[w], idx_v)
        pltpu.sync_copy(pidx.at[w], pos_v)

        def body(s, carry):
            pltpu.async_copy(tab.at[idx_v.at[s]], buf_v, sem).wait()
            pltpu.async_copy(buf_v, out.at[pos_v.at[s]], sem).wait()
            return carry

        lax.fori_loop(0, 4, body, 0)

    return k(table, rowsk3d, pos3d)


def _cand_positions():
    q = jnp.arange(32 * HB, dtype=jnp.int32)
    return ((q & (HB - 1)) * 32 + (q >> 9)).reshape(NW, 4, 128)


def _nce_gather(table_flat, idx_flat):
    @functools.partial(
        pl.kernel,
        out_type=jax.ShapeDtypeStruct((HB, KP), jnp.float32),
        mesh=_sc_mesh(),
        scratch_types=[
            pltpu.VMEM((KP,), jnp.int32),
            pltpu.VMEM((NADDR, 128), jnp.int32),
            pltpu.VMEM((KP,), jnp.float32),
            pltpu.SemaphoreType.DMA,
        ],
    )
    def k(tab, idx_hbm, out, col_v, addr_v, val_v, sem):
        w = _wid()

        def row_body(r, carry):
            b = w * RPW_H + r
            pltpu.sync_copy(idx_hbm.at[pl.ds(b * KP, KP)], col_v)

            def addr_chunk(j, c2):
                for o in range(8):
                    col = col_v[pl.ds(j * 128 + o * 16, 16)]
                    t = lax.shift_right_arithmetic(col, 11)
                    cc = lax.bitwise_and(col, TN - 1)
                    f = lax.shift_left(t, 20) + (b * TN + cc)
                    addr_v[j, pl.ds(o * 16, 16)] = f
                return c2

            lax.fori_loop(0, NADDR, addr_chunk, 0)

            def fire(j, c2):
                pltpu.async_copy(
                    tab.at[addr_v.at[j]], val_v.at[pl.ds(j * 128, 128)], sem)
                return c2

            lax.fori_loop(0, NADDR, fire, 0)

            def drain(j, c2):
                pltpu.make_async_copy(
                    tab.at[addr_v.at[j]], val_v.at[pl.ds(j * 128, 128)], sem
                ).wait()
                return c2

            lax.fori_loop(0, NADDR, drain, 0)
            pltpu.sync_copy(val_v, out.at[b])
            return carry

        lax.fori_loop(0, RPW_H, row_body, 0)

    return k(table_flat, idx_flat)


def _label_gather(trainLabel, yi3d):
    @functools.partial(
        pl.kernel,
        out_type=jax.ShapeDtypeStruct((BS * 32,), jnp.int32),
        mesh=_sc_mesh(),
        scratch_types=[
            pltpu.VMEM((8, 128), jnp.int32),
            pltpu.VMEM((128,), jnp.int32),
            pltpu.SemaphoreType.DMA,
        ],
    )
    def k(tab, ridx, out, idx_v, buf_v, sem):
        w = _wid()
        pltpu.sync_copy(ridx.at[w], idx_v)

        def body(s, carry):
            pltpu.async_copy(tab.at[idx_v.at[s]], buf_v, sem).wait()
            pltpu.sync_copy(buf_v, out.at[pl.ds(w * 1024 + s * 128, 128)])
            return carry

        lax.fori_loop(0, 8, body, 0)

    return k(trainLabel, yi3d)


def _update_gathers(memory, x, ysort, wsort):
    @functools.partial(
        pl.kernel,
        out_type=[
            jax.ShapeDtypeStruct((BS, IN), jnp.float32),
            jax.ShapeDtypeStruct((BS, IN), jnp.float32),
        ],
        mesh=_sc_mesh(),
        scratch_types=[
            pltpu.VMEM((ROWS_PER_W,), jnp.int32),
            pltpu.VMEM((ROWS_PER_W, IN), jnp.float32),
            pltpu.SemaphoreType.DMA,
        ],
    )
    def k(mem, xx, ys, ws, out_my, out_xw, idx_v, buf_v, sem):
        w = _wid()
        base = w * ROWS_PER_W
        pltpu.sync_copy(ys.at[pl.ds(base, ROWS_PER_W)], idx_v)
        pltpu.async_copy(mem.at[idx_v], buf_v, sem).wait()
        pltpu.sync_copy(buf_v, out_my.at[pl.ds(base, ROWS_PER_W)])
        pltpu.sync_copy(ws.at[pl.ds(base, ROWS_PER_W)], idx_v)
        pltpu.async_copy(xx.at[idx_v], buf_v, sem).wait()
        pltpu.sync_copy(buf_v, out_xw.at[pl.ds(base, ROWS_PER_W)])

    return k(memory, x, ysort, wsort)


def _update_scatter(new_mem_ref, normed, y):
    @functools.partial(
        pl.kernel,
        out_type=(),
        mesh=_sc_mesh(),
        scratch_types=[
            pltpu.VMEM((ROWS_PER_W,), jnp.int32),
            pltpu.VMEM((ROWS_PER_W, IN), jnp.float32),
            pltpu.SemaphoreType.DMA,
        ],
    )
    def k(nrm, yy, out, idx_v, buf_v, sem):
        w = _wid()
        base = w * ROWS_PER_W
        pltpu.sync_copy(yy.at[pl.ds(base, ROWS_PER_W)], idx_v)
        pltpu.sync_copy(nrm.at[pl.ds(base, ROWS_PER_W)], buf_v)
        pltpu.async_copy(buf_v, out.at[idx_v], sem).wait()

    k(normed, y, new_mem_ref)


def _half_pipeline(xh, idx_flat, memory, harange):
    out_h, cmax = _similarity(xh, memory)

    nce_pad = _nce_gather(out_h.reshape(FLAT_H), idx_flat)

    chunk_ids, rowsk = _topchunks(cmax)
    cid_t = chunk_ids.T

    cand = _cand_gather(out_h, rowsk.reshape(NW, 4, 128), _cand_positions())
    cand = cand.reshape(HB, 32 * 128)
    cols = (cid_t[:, :, None] * 128
            + jnp.arange(128, dtype=jnp.int32)[None, None, :]).reshape(HB, 32 * 128)
    return nce_pad, cand, cols


def kernel(x, target, y, idx, trainLabel, memory):
    iarange = jnp.arange(BS, dtype=jnp.int32)
    winner = jnp.argmax(jnp.where(y[None, :] == y[:, None], iarange[None, :], -1),
                        axis=1).astype(jnp.int32)

    mem_y, xw = _update_gathers(memory, x, y, winner)
    normed = _mix_norm(mem_y, xw)
    mem_for_clone, _ = jax.lax.optimization_barrier((memory, normed))
    new_mem_ref = jax.new_ref(mem_for_clone)
    _update_scatter(new_mem_ref, normed, y)
    new_memory = new_mem_ref[...]

    idx_pad = jnp.pad(idx, ((0, 0), (0, KP - (K + 1)))).reshape(BS * KP)
    idx_flatA = idx_pad[:HB * KP]
    idx_flatB = idx_pad[HB * KP:]

    harange = iarange[:HB]
    nceA, candA, colsA = _half_pipeline(x[:HB], idx_flatA, memory, harange)
    nceB, candB, colsB = _half_pipeline(x[HB:], idx_flatB, memory, harange)

    ydA, yiA = _topk(candA, colsA)
    ydB, yiB = _topk(candB, colsB)
    yd = jnp.concatenate([ydA, ydB], axis=0)
    yi = jnp.concatenate([yiA, yiB], axis=0)
    retrieval = _label_gather(trainLabel, yi.reshape(NW, 8, 128)).reshape(BS, 32)

    eA, rsA = _exp_norm(nceA)
    eB, rsB = _exp_norm(nceB)
    total = jnp.sum(rsA) + jnp.sum(rsB)
    Z = total / jnp.float32(BS * (K + 1)) * jnp.float32(OUT)
    out = jnp.concatenate([eA[:, :K + 1], eB[:, :K + 1]], axis=0) / Z
    probs = (jnp.sum(eA[:, 0] / rsA[:, 0]) + jnp.sum(eB[:, 0] / rsB[:, 0])) / jnp.float32(BS)

    return out, probs, yd, retrieval, new_memory


# speedup vs baseline: 1.0435x; 1.0286x over previous
import functools

import jax
import jax.numpy as jnp
from jax import lax
from jax.experimental import pallas as pl
from jax.experimental.pallas import tpu as pltpu
from jax.experimental.pallas import tpu_sc as plsc

BS = 1024
IN = 128
OUT = 100000
K = 4096
T = 0.07
MOMENTUM = 0.5

HB = 512
TN = 2048
NT = 49
NCHT = TN // 128
NCH = NT * NCHT
KP = 33 * 128
NEG = -1e30


def _sim_body(x_ref, m_ref, out_ref, cmax_ref):
    t = pl.program_id(0)
    tile = jax.lax.dot_general(
        x_ref[...], m_ref[...], (((1,), (1,)), ((), ())),
        preferred_element_type=jnp.float32,
        precision=jax.lax.Precision.DEFAULT)
    col = jax.lax.broadcasted_iota(jnp.int32, (HB, TN), 1) + t * TN
    tile = jnp.where(col < OUT, tile, NEG)
    out_ref[...] = tile.reshape(HB * NCHT, 128)
    for c in range(NCHT):
        cmax_ref[0, c, :] = jnp.max(tile[:, c * 128:(c + 1) * 128], axis=1)


def _similarity(xh, memory):
    return pl.pallas_call(
        _sim_body,
        grid=(NT,),
        in_specs=[
            pl.BlockSpec((HB, IN), lambda t: (0, 0)),
            pl.BlockSpec((TN, IN), lambda t: (t, 0)),
        ],
        out_specs=[
            pl.BlockSpec((HB * NCHT, 128), lambda t: (t, 0)),
            pl.BlockSpec((1, NCHT, HB), lambda t: (t, 0, 0)),
        ],
        out_shape=[
            jax.ShapeDtypeStruct((NT * HB * NCHT, 128), jnp.float32),
            jax.ShapeDtypeStruct((NT, NCHT, HB), jnp.float32),
        ],
    )(xh, memory)


def _topchunk_body(cm_ref, cid_ref, rows_ref):
    v = cm_ref[...].reshape(NCH, HB)
    ii = jax.lax.broadcasted_iota(jnp.int32, (NCH, HB), 0)
    bvec = jax.lax.broadcasted_iota(jnp.int32, (NCH, HB), 1)[0]
    for k in range(32):
        m = jnp.max(v, axis=0)
        sel = jnp.min(jnp.where(v == m[None, :], ii, NCH), axis=0)
        cid_ref[k, :] = sel
        rows_ref[k, :] = ((jax.lax.shift_right_logical(sel, 4) * HB + bvec) * NCHT
                          + jax.lax.bitwise_and(sel, NCHT - 1))
        v = jnp.where(ii == sel[None, :], -jnp.inf, v)


def _topchunks(cmax):
    return pl.pallas_call(
        _topchunk_body,
        out_shape=[
            jax.ShapeDtypeStruct((32, HB), jnp.int32),
            jax.ShapeDtypeStruct((32, HB), jnp.int32),
        ],
    )(cmax)


def _topk_body(cand_ref, cols_ref, yd_ref, yi_ref):
    v = cand_ref[...]
    cols = cols_ref[...]
    for k in range(32):
        m = jnp.max(v, axis=1)
        sel = jnp.min(jnp.where(v == m[:, None], cols, jnp.int32(2**30)), axis=1)
        yd_ref[:, k] = m
        yi_ref[:, k] = sel
        v = jnp.where(cols == sel[:, None], -jnp.inf, v)


def _topk(cand, cols):
    return pl.pallas_call(
        _topk_body,
        out_shape=[
            jax.ShapeDtypeStruct((HB, 32), jnp.float32),
            jax.ShapeDtypeStruct((HB, 32), jnp.int32),
        ],
    )(cand, cols)


def _exp_body(nce_ref, e_ref, rs_ref):
    col = jax.lax.broadcasted_iota(jnp.int32, (HB, KP), 1)
    v = jnp.where(col <= K, nce_ref[...], -jnp.inf)
    e = jnp.exp(v * jnp.float32(1.0 / T))
    e_ref[...] = e
    rs_ref[...] = jnp.sum(e, axis=1, keepdims=True)


def _exp_norm(nce_pad):
    return pl.pallas_call(
        _exp_body,
        out_shape=[
            jax.ShapeDtypeStruct((HB, KP), jnp.float32),
            jax.ShapeDtypeStruct((HB, 1), jnp.float32),
        ],
    )(nce_pad)


def _norm_body(my_ref, xw_ref, o_ref):
    w = my_ref[...] * jnp.float32(MOMENTUM) + xw_ref[...] * jnp.float32(1.0 - MOMENTUM)
    n = jnp.maximum(jnp.sqrt(jnp.sum(w * w, axis=1, keepdims=True)), 1e-12)
    o_ref[...] = w / n


def _mix_norm(mem_y, xw):
    return pl.pallas_call(
        _norm_body,
        out_shape=jax.ShapeDtypeStruct((BS, IN), jnp.float32),
    )(mem_y, xw)


NW = 32
ROWS_PER_W = BS // NW
RPW_H = HB // NW
FLAT_H = NT * HB * TN
NADDR = KP // 128


def _sc_mesh():
    return plsc.VectorSubcoreMesh(core_axis_name="c", subcore_axis_name="s")


def _wid():
    return lax.axis_index("s") * 2 + lax.axis_index("c")


def _cand_gather(table, rowsk3d, pos3d):
    @functools.partial(
        pl.kernel,
        out_type=jax.ShapeDtypeStruct((HB * 32, 128), jnp.float32),
        mesh=_sc_mesh(),
        scratch_types=[
            pltpu.VMEM((4, 128), jnp.int32),
            pltpu.VMEM((4, 128), jnp.int32),
            pltpu.VMEM((128, 128), jnp.float32),
            pltpu.SemaphoreType.DMA,
        ],
    )
    def k(tab, ridx, pidx, out, idx_v, pos_v, buf_v, sem):
        w = _wid()
        pltpu.sync_copy(ridx.at[w], idx_v)
        pltpu.sync_copy(pidx.at[w], pos_v)

        def body(s, carry):
            pltpu.async_copy(tab.at[idx_v.at[s]], buf_v, sem).wait()
            pltpu.async_copy(buf_v, out.at[pos_v.at[s]], sem).wait()
            return carry

        lax.fori_loop(0, 4, body, 0)

    return k(table, rowsk3d, pos3d)


def _cand_positions():
    q = jnp.arange(32 * HB, dtype=jnp.int32)
    return ((q & (HB - 1)) * 32 + (q >> 9)).reshape(NW, 4, 128)


def _nce_gather(table_flat, idx_flat):
    @functools.partial(
        pl.kernel,
        out_type=jax.ShapeDtypeStruct((HB, KP), jnp.float32),
        mesh=_sc_mesh(),
        scratch_types=[
            pltpu.VMEM((KP,), jnp.int32),
            pltpu.VMEM((NADDR, 128), jnp.int32),
            pltpu.VMEM((KP,), jnp.float32),
            pltpu.SemaphoreType.DMA,
        ],
    )
    def k(tab, idx_hbm, out, col_v, addr_v, val_v, sem):
        w = _wid()

        def row_body(r, carry):
            b = w * RPW_H + r
            pltpu.sync_copy(idx_hbm.at[pl.ds(b * KP, KP)], col_v)

            def addr_chunk(j, c2):
                for o in range(8):
                    col = col_v[pl.ds(j * 128 + o * 16, 16)]
                    t = lax.shift_right_arithmetic(col, 11)
                    cc = lax.bitwise_and(col, TN - 1)
                    f = lax.shift_left(t, 20) + (b * TN + cc)
                    addr_v[j, pl.ds(o * 16, 16)] = f
                return c2

            lax.fori_loop(0, NADDR, addr_chunk, 0)

            def fire(j, c2):
                pltpu.async_copy(
                    tab.at[addr_v.at[j]], val_v.at[pl.ds(j * 128, 128)], sem)
                return c2

            lax.fori_loop(0, NADDR, fire, 0)

            def drain(j, c2):
                pltpu.make_async_copy(
                    tab.at[addr_v.at[j]], val_v.at[pl.ds(j * 128, 128)], sem
                ).wait()
                return c2

            lax.fori_loop(0, NADDR, drain, 0)
            pltpu.sync_copy(val_v, out.at[b])
            return carry

        lax.fori_loop(0, RPW_H, row_body, 0)

    return k(table_flat, idx_flat)


def _label_gather(trainLabel, yi3d):
    @functools.partial(
        pl.kernel,
        out_type=jax.ShapeDtypeStruct((BS * 32,), jnp.int32),
        mesh=_sc_mesh(),
        scratch_types=[
            pltpu.VMEM((8, 128), jnp.int32),
            pltpu.VMEM((128,), jnp.int32),
            pltpu.SemaphoreType.DMA,
        ],
    )
    def k(tab, ridx, out, idx_v, buf_v, sem):
        w = _wid()
        pltpu.sync_copy(ridx.at[w], idx_v)

        def body(s, carry):
            pltpu.async_copy(tab.at[idx_v.at[s]], buf_v, sem).wait()
            pltpu.sync_copy(buf_v, out.at[pl.ds(w * 1024 + s * 128, 128)])
            return carry

        lax.fori_loop(0, 8, body, 0)

    return k(trainLabel, yi3d)


def _update_gathers(memory, x, ysort, wsort):
    @functools.partial(
        pl.kernel,
        out_type=[
            jax.ShapeDtypeStruct((BS, IN), jnp.float32),
            jax.ShapeDtypeStruct((BS, IN), jnp.float32),
        ],
        mesh=_sc_mesh(),
        scratch_types=[
            pltpu.VMEM((ROWS_PER_W,), jnp.int32),
            pltpu.VMEM((ROWS_PER_W, IN), jnp.float32),
            pltpu.SemaphoreType.DMA,
        ],
    )
    def k(mem, xx, ys, ws, out_my, out_xw, idx_v, buf_v, sem):
        w = _wid()
        base = w * ROWS_PER_W
        pltpu.sync_copy(ys.at[pl.ds(base, ROWS_PER_W)], idx_v)
        pltpu.async_copy(mem.at[idx_v], buf_v, sem).wait()
        pltpu.sync_copy(buf_v, out_my.at[pl.ds(base, ROWS_PER_W)])
        pltpu.sync_copy(ws.at[pl.ds(base, ROWS_PER_W)], idx_v)
        pltpu.async_copy(xx.at[idx_v], buf_v, sem).wait()
        pltpu.sync_copy(buf_v, out_xw.at[pl.ds(base, ROWS_PER_W)])

    return k(memory, x, ysort, wsort)


def _update_scatter(new_mem_ref, normed, y):
    @functools.partial(
        pl.kernel,
        out_type=(),
        mesh=_sc_mesh(),
        scratch_types=[
            pltpu.VMEM((ROWS_PER_W,), jnp.int32),
            pltpu.VMEM((ROWS_PER_W, IN), jnp.float32),
            pltpu.SemaphoreType.DMA,
        ],
    )
    def k(nrm, yy, out, idx_v, buf_v, sem):
        w = _wid()
        base = w * ROWS_PER_W
        pltpu.sync_copy(yy.at[pl.ds(base, ROWS_PER_W)], idx_v)
        pltpu.sync_copy(nrm.at[pl.ds(base, ROWS_PER_W)], buf_v)
        pltpu.async_copy(buf_v, out.at[idx_v], sem).wait()

    k(normed, y, new_mem_ref)


def _half_pipeline(xh, idx_flat, memory, harange, after=None):
    out_h, cmax = _similarity(xh, memory)

    nce_pad = _nce_gather(out_h.reshape(FLAT_H), idx_flat)

    if after is not None:
        cmax, _ = jax.lax.optimization_barrier((cmax, after))
    chunk_ids, rowsk = _topchunks(cmax)
    cid_t = chunk_ids.T

    cand = _cand_gather(out_h, rowsk.reshape(NW, 4, 128), _cand_positions())
    cand = cand.reshape(HB, 32 * 128)
    cols = (cid_t[:, :, None] * 128
            + jnp.arange(128, dtype=jnp.int32)[None, None, :]).reshape(HB, 32 * 128)
    return nce_pad, cand, cols


def kernel(x, target, y, idx, trainLabel, memory):
    iarange = jnp.arange(BS, dtype=jnp.int32)
    winner = jnp.argmax(jnp.where(y[None, :] == y[:, None], iarange[None, :], -1),
                        axis=1).astype(jnp.int32)

    mem_y, xw = _update_gathers(memory, x, y, winner)
    normed = _mix_norm(mem_y, xw)
    mem_for_clone, _ = jax.lax.optimization_barrier((memory, normed))
    new_mem_ref = jax.new_ref(mem_for_clone)
    _update_scatter(new_mem_ref, normed, y)
    new_memory = new_mem_ref[...]

    idx_pad = jnp.pad(idx, ((0, 0), (0, KP - (K + 1)))).reshape(BS * KP)
    idx_flatA = idx_pad[:HB * KP]
    idx_flatB = idx_pad[HB * KP:]

    harange = iarange[:HB]
    nceA, candA, colsA = _half_pipeline(x[:HB], idx_flatA, memory, harange)
    nceB, candB, colsB = _half_pipeline(x[HB:], idx_flatB, memory, harange,
                                        after=nceA)

    ydA, yiA = _topk(candA, colsA)
    ydB, yiB = _topk(candB, colsB)
    yd = jnp.concatenate([ydA, ydB], axis=0)
    yi = jnp.concatenate([yiA, yiB], axis=0)
    retrieval = _label_gather(trainLabel, yi.reshape(NW, 8, 128)).reshape(BS, 32)

    eA, rsA = _exp_norm(nceA)
    eB, rsB = _exp_norm(nceB)
    total = jnp.sum(rsA) + jnp.sum(rsB)
    Z = total / jnp.float32(BS * (K + 1)) * jnp.float32(OUT)
    out = jnp.concatenate([eA[:, :K + 1], eB[:, :K + 1]], axis=0) / Z
    probs = (jnp.sum(eA[:, 0] / rsA[:, 0]) + jnp.sum(eB[:, 0] / rsB[:, 0])) / jnp.float32(BS)

    return out, probs, yd, retrieval, new_memory


# speedup vs baseline: 1.0785x; 1.0336x over previous
import functools

import jax
import jax.numpy as jnp
from jax import lax
from jax.experimental import pallas as pl
from jax.experimental.pallas import tpu as pltpu
from jax.experimental.pallas import tpu_sc as plsc

BS = 1024
IN = 128
OUT = 100000
K = 4096
T = 0.07
MOMENTUM = 0.5

HB = 512
TN = 2048
NT = 49
NCHT = TN // 128
NCH = NT * NCHT
KP = 33 * 128
NEG = -1e30


def _sim_body(x_ref, m_ref, out_ref, cmax_ref):
    t = pl.program_id(0)
    tile = jax.lax.dot_general(
        x_ref[...], m_ref[...], (((1,), (1,)), ((), ())),
        preferred_element_type=jnp.float32,
        precision=jax.lax.Precision.DEFAULT)
    col = jax.lax.broadcasted_iota(jnp.int32, (HB, TN), 1) + t * TN
    tile = jnp.where(col < OUT, tile, NEG)
    out_ref[...] = tile.reshape(HB * NCHT, 128)
    for c in range(NCHT):
        cmax_ref[0, c, :] = jnp.max(tile[:, c * 128:(c + 1) * 128], axis=1)


def _similarity(xh, memory):
    return pl.pallas_call(
        _sim_body,
        grid=(NT,),
        in_specs=[
            pl.BlockSpec((HB, IN), lambda t: (0, 0)),
            pl.BlockSpec((TN, IN), lambda t: (t, 0)),
        ],
        out_specs=[
            pl.BlockSpec((HB * NCHT, 128), lambda t: (t, 0)),
            pl.BlockSpec((1, NCHT, HB), lambda t: (t, 0, 0)),
        ],
        out_shape=[
            jax.ShapeDtypeStruct((NT * HB * NCHT, 128), jnp.float32),
            jax.ShapeDtypeStruct((NT, NCHT, HB), jnp.float32),
        ],
    )(xh, memory)


def _topchunk_body(cm_ref, cid_ref, rows_ref):
    v = cm_ref[...].reshape(NCH, HB)
    ii = jax.lax.broadcasted_iota(jnp.int32, (NCH, HB), 0)
    bvec = jax.lax.broadcasted_iota(jnp.int32, (NCH, HB), 1)[0]
    for k in range(32):
        m = jnp.max(v, axis=0)
        sel = jnp.min(jnp.where(v == m[None, :], ii, NCH), axis=0)
        cid_ref[k, :] = sel
        rows_ref[k, :] = ((jax.lax.shift_right_logical(sel, 4) * HB + bvec) * NCHT
                          + jax.lax.bitwise_and(sel, NCHT - 1))
        v = jnp.where(ii == sel[None, :], -jnp.inf, v)


def _topchunks(cmax):
    return pl.pallas_call(
        _topchunk_body,
        out_shape=[
            jax.ShapeDtypeStruct((32, HB), jnp.int32),
            jax.ShapeDtypeStruct((32, HB), jnp.int32),
        ],
    )(cmax)


def _topk_body(cand_ref, cols_ref, yd_ref, yi_ref):
    v = cand_ref[...]
    cols = cols_ref[...]
    for k in range(32):
        m = jnp.max(v, axis=1)
        sel = jnp.min(jnp.where(v == m[:, None], cols, jnp.int32(2**30)), axis=1)
        yd_ref[:, k] = m
        yi_ref[:, k] = sel
        v = jnp.where(cols == sel[:, None], -jnp.inf, v)


def _topk(cand, cols):
    return pl.pallas_call(
        _topk_body,
        out_shape=[
            jax.ShapeDtypeStruct((HB, 32), jnp.float32),
            jax.ShapeDtypeStruct((HB, 32), jnp.int32),
        ],
    )(cand, cols)


def _exp_body(nce_ref, e_ref, rs_ref):
    col = jax.lax.broadcasted_iota(jnp.int32, (HB, KP), 1)
    v = jnp.where(col <= K, nce_ref[...], -jnp.inf)
    e = jnp.exp(v * jnp.float32(1.0 / T))
    e_ref[...] = e
    rs_ref[...] = jnp.sum(e, axis=1, keepdims=True)


def _exp_norm(nce_pad):
    return pl.pallas_call(
        _exp_body,
        out_shape=[
            jax.ShapeDtypeStruct((HB, KP), jnp.float32),
            jax.ShapeDtypeStruct((HB, 1), jnp.float32),
        ],
    )(nce_pad)


def _norm_body(my_ref, xw_ref, o_ref):
    w = my_ref[...] * jnp.float32(MOMENTUM) + xw_ref[...] * jnp.float32(1.0 - MOMENTUM)
    n = jnp.maximum(jnp.sqrt(jnp.sum(w * w, axis=1, keepdims=True)), 1e-12)
    o_ref[...] = w / n


def _mix_norm(mem_y, xw):
    return pl.pallas_call(
        _norm_body,
        out_shape=jax.ShapeDtypeStruct((BS, IN), jnp.float32),
    )(mem_y, xw)


NW = 32
ROWS_PER_W = BS // NW
RPW_H = HB // NW
FLAT_H = NT * HB * TN
NADDR = KP // 128


def _sc_mesh():
    return plsc.VectorSubcoreMesh(core_axis_name="c", subcore_axis_name="s")


def _wid():
    return lax.axis_index("s") * 2 + lax.axis_index("c")


def _cand_gather(table, rowsk3d, pos3d):
    @functools.partial(
        pl.kernel,
        out_type=jax.ShapeDtypeStruct((HB * 32, 128), jnp.float32),
        mesh=_sc_mesh(),
        scratch_types=[
            pltpu.VMEM((4, 128), jnp.int32),
            pltpu.VMEM((4, 128), jnp.int32),
            pltpu.VMEM((128, 128), jnp.float32),
            pltpu.SemaphoreType.DMA,
        ],
    )
    def k(tab, ridx, pidx, out, idx_v, pos_v, buf_v, sem):
        w = _wid()
        pltpu.sync_copy(ridx.at[w], idx_v)
        pltpu.sync_copy(pidx.at[w], pos_v)

        def body(s, carry):
            pltpu.async_copy(tab.at[idx_v.at[s]], buf_v, sem).wait()
            pltpu.async_copy(buf_v, out.at[pos_v.at[s]], sem).wait()
            return carry

        lax.fori_loop(0, 4, body, 0)

    return k(table, rowsk3d, pos3d)


def _cand_positions():
    q = jnp.arange(32 * HB, dtype=jnp.int32)
    return ((q & (HB - 1)) * 32 + (q >> 9)).reshape(NW, 4, 128)


def _nce_gather(table_flat, idx_flat, base):
    @functools.partial(
        pl.kernel,
        out_type=jax.ShapeDtypeStruct((HB, KP), jnp.float32),
        mesh=_sc_mesh(),
        scratch_types=[
            pltpu.VMEM((KP,), jnp.int32),
            pltpu.VMEM((NADDR, 128), jnp.int32),
            pltpu.VMEM((KP,), jnp.float32),
            pltpu.SemaphoreType.DMA,
        ],
    )
    def k(tab, idx_hbm, out, col_v, addr_v, val_v, sem):
        w = _wid()

        def row_body(r, carry):
            b = w * RPW_H + r
            pltpu.sync_copy(idx_hbm.at[pl.ds((base + b) * KP, KP)], col_v)

            def addr_chunk(j, c2):
                for o in range(8):
                    col = col_v[pl.ds(j * 128 + o * 16, 16)]
                    t = lax.shift_right_arithmetic(col, 11)
                    cc = lax.bitwise_and(col, TN - 1)
                    f = lax.shift_left(t, 20) + (b * TN + cc)
                    addr_v[j, pl.ds(o * 16, 16)] = f
                return c2

            lax.fori_loop(0, NADDR, addr_chunk, 0)

            def fire(j, c2):
                pltpu.async_copy(
                    tab.at[addr_v.at[j]], val_v.at[pl.ds(j * 128, 128)], sem)
                return c2

            lax.fori_loop(0, NADDR, fire, 0)

            def drain(j, c2):
                pltpu.make_async_copy(
                    tab.at[addr_v.at[j]], val_v.at[pl.ds(j * 128, 128)], sem
                ).wait()
                return c2

            lax.fori_loop(0, NADDR, drain, 0)
            pltpu.sync_copy(val_v, out.at[b])
            return carry

        lax.fori_loop(0, RPW_H, row_body, 0)

    return k(table_flat, idx_flat)


def _label_gather(trainLabel, yi3d):
    @functools.partial(
        pl.kernel,
        out_type=jax.ShapeDtypeStruct((BS * 32,), jnp.int32),
        mesh=_sc_mesh(),
        scratch_types=[
            pltpu.VMEM((8, 128), jnp.int32),
            pltpu.VMEM((128,), jnp.int32),
            pltpu.SemaphoreType.DMA,
        ],
    )
    def k(tab, ridx, out, idx_v, buf_v, sem):
        w = _wid()
        pltpu.sync_copy(ridx.at[w], idx_v)

        def body(s, carry):
            pltpu.async_copy(tab.at[idx_v.at[s]], buf_v, sem).wait()
            pltpu.sync_copy(buf_v, out.at[pl.ds(w * 1024 + s * 128, 128)])
            return carry

        lax.fori_loop(0, 8, body, 0)

    return k(trainLabel, yi3d)


def _update_gathers(memory, x, ysort, wsort):
    @functools.partial(
        pl.kernel,
        out_type=[
            jax.ShapeDtypeStruct((BS, IN), jnp.float32),
            jax.ShapeDtypeStruct((BS, IN), jnp.float32),
        ],
        mesh=_sc_mesh(),
        scratch_types=[
            pltpu.VMEM((ROWS_PER_W,), jnp.int32),
            pltpu.VMEM((ROWS_PER_W, IN), jnp.float32),
            pltpu.SemaphoreType.DMA,
        ],
    )
    def k(mem, xx, ys, ws, out_my, out_xw, idx_v, buf_v, sem):
        w = _wid()
        base = w * ROWS_PER_W
        pltpu.sync_copy(ys.at[pl.ds(base, ROWS_PER_W)], idx_v)
        pltpu.async_copy(mem.at[idx_v], buf_v, sem).wait()
        pltpu.sync_copy(buf_v, out_my.at[pl.ds(base, ROWS_PER_W)])
        pltpu.sync_copy(ws.at[pl.ds(base, ROWS_PER_W)], idx_v)
        pltpu.async_copy(xx.at[idx_v], buf_v, sem).wait()
        pltpu.sync_copy(buf_v, out_xw.at[pl.ds(base, ROWS_PER_W)])

    return k(memory, x, ysort, wsort)


def _update_scatter(new_mem_ref, normed, y):
    @functools.partial(
        pl.kernel,
        out_type=(),
        mesh=_sc_mesh(),
        scratch_types=[
            pltpu.VMEM((ROWS_PER_W,), jnp.int32),
            pltpu.VMEM((ROWS_PER_W, IN), jnp.float32),
            pltpu.SemaphoreType.DMA,
        ],
    )
    def k(nrm, yy, out, idx_v, buf_v, sem):
        w = _wid()
        base = w * ROWS_PER_W
        pltpu.sync_copy(yy.at[pl.ds(base, ROWS_PER_W)], idx_v)
        pltpu.sync_copy(nrm.at[pl.ds(base, ROWS_PER_W)], buf_v)
        pltpu.async_copy(buf_v, out.at[idx_v], sem).wait()

    k(normed, y, new_mem_ref)


def _half_pipeline(xh, idx_flat, memory, harange, base, after=None):
    out_h, cmax = _similarity(xh, memory)

    nce_pad = _nce_gather(out_h.reshape(FLAT_H), idx_flat, base)

    if after is not None:
        cmax, _ = jax.lax.optimization_barrier((cmax, after))
    chunk_ids, rowsk = _topchunks(cmax)
    cid_t = chunk_ids.T

    cand = _cand_gather(out_h, rowsk.reshape(NW, 4, 128), _cand_positions())
    cand = cand.reshape(HB, 32 * 128)
    cols = (cid_t[:, :, None] * 128
            + jnp.arange(128, dtype=jnp.int32)[None, None, :]).reshape(HB, 32 * 128)
    return nce_pad, cand, cols


def kernel(x, target, y, idx, trainLabel, memory):
    iarange = jnp.arange(BS, dtype=jnp.int32)
    winner = jnp.argmax(jnp.where(y[None, :] == y[:, None], iarange[None, :], -1),
                        axis=1).astype(jnp.int32)

    mem_y, xw = _update_gathers(memory, x, y, winner)
    normed = _mix_norm(mem_y, xw)
    mem_for_clone, _ = jax.lax.optimization_barrier((memory, normed))
    new_mem_ref = jax.new_ref(mem_for_clone)
    _update_scatter(new_mem_ref, normed, y)
    new_memory = new_mem_ref[...]

    idx_pad = jnp.pad(idx, ((0, 0), (0, KP - (K + 1)))).reshape(BS * KP)

    harange = iarange[:HB]
    nceA, candA, colsA = _half_pipeline(x[:HB], idx_pad, memory, harange, 0)
    nceB, candB, colsB = _half_pipeline(x[HB:], idx_pad, memory, harange, HB,
                                        after=nceA)

    ydA, yiA = _topk(candA, colsA)
    ydB, yiB = _topk(candB, colsB)
    yd = jnp.concatenate([ydA, ydB], axis=0)
    yi = jnp.concatenate([yiA, yiB], axis=0)
    retrieval = _label_gather(trainLabel, yi.reshape(NW, 8, 128)).reshape(BS, 32)

    eA, rsA = _exp_norm(nceA)
    eB, rsB = _exp_norm(nceB)
    total = jnp.sum(rsA) + jnp.sum(rsB)
    Z = total / jnp.float32(BS * (K + 1)) * jnp.float32(OUT)
    out = jnp.concatenate([eA[:, :K + 1], eB[:, :K + 1]], axis=0) / Z
    probs = (jnp.sum(eA[:, 0] / rsA[:, 0]) + jnp.sum(eB[:, 0] / rsB[:, 0])) / jnp.float32(BS)

    return out, probs, yd, retrieval, new_memory
